# baseline (speedup 1.0000x reference)
import numpy as np
import jax
import jax.numpy as jnp
from jax import lax
from jax.experimental import pallas as pl
from jax.experimental.pallas import tpu as pltpu

_D = 32
_G = 512
_NVOC = 184
_EVOC = 24
_NODE_OFF = (0, 120, 125, 137, 149, 159, 165, 172, 174)
_EDGE_OFF = (0, 15, 21)
_NODE_PAD_IDX = 176
_EDGE_PAD_IDX = 23

_C00 = (((0,), (0,)), ((), ()))


def _branch_kernel(idx_ref, ntab_ref, etab_ref, w_ref, v_ref, o_ref):
    n = idx_ref.shape[1]
    e = n
    g = o_ref.shape[0]
    idx = idx_ref[...]
    w = w_ref[...]
    v = v_ref[...]
    n_layers = v.shape[0]

    iota_nv = lax.broadcasted_iota(jnp.int32, (_NVOC, n), 0)
    nht = (iota_nv == idx[0:1, :]).astype(jnp.float32)
    for i in range(1, 9):
        nht = nht + (iota_nv == idx[i:i + 1, :]).astype(jnp.float32)
    x2 = lax.dot_general(nht, ntab_ref[...], _C00,
                         preferred_element_type=jnp.float32)
    x = x2[:, :_D] + x2[:, _D:]

    iota_ev = lax.broadcasted_iota(jnp.int32, (_EVOC, e), 0)
    eht = (iota_ev == idx[9:10, :]).astype(jnp.float32)
    for i in range(10, 12):
        eht = eht + (iota_ev == idx[i:i + 1, :]).astype(jnp.float32)
    ea2 = lax.dot_general(eht, etab_ref[...], _C00,
                          preferred_element_type=jnp.float32)
    ea = ea2[:, :_D] + ea2[:, _D:]

    iota_ne = lax.broadcasted_iota(jnp.int32, (n, e), 0)
    ost = (iota_ne == idx[12:13, :]).astype(jnp.float32)
    odt = (iota_ne == idx[13:14, :]).astype(jnp.float32)
    iota_gn = lax.broadcasted_iota(jnp.int32, (g, n), 0)
    pm = (iota_gn == idx[14:15, :]).astype(jnp.float32)
    pm = pm / jnp.maximum(jnp.sum(pm, axis=1, keepdims=True), 1.0)

    for l in range(n_layers):
        xg = lax.dot_general(ost, x, _C00,
                             preferred_element_type=jnp.float32)
        msg = jnp.maximum(xg + ea, 0.0)
        agg = jnp.dot(odt, msg, preferred_element_type=jnp.float32)
        h = x + agg
        h1 = jnp.maximum(
            jnp.dot(h, w[2 * l], preferred_element_type=jnp.float32)
            + v[l, 0:1], 0.0)
        h2 = jnp.dot(h1, w[2 * l + 1],
                     preferred_element_type=jnp.float32) + v[l, 1:2]
        x = jnp.maximum(h2 * v[l, 2:3] + v[l, 3:4], 0.0)

    o_ref[...] = jnp.dot(pm, x, preferred_element_type=jnp.float32)


def _edge_agg_kernel(e_ref, xb_ref, o_ref):
    eh = e_ref.shape[1]
    nfull = xb_ref.shape[0]
    nout = o_ref.shape[0]
    d2 = xb_ref.shape[1]
    be = 1024 if eh % 1024 == 0 else eh
    xb = xb_ref[...]
    iota_g = lax.broadcasted_iota(jnp.int32, (nfull, be), 0)
    iota_s = lax.broadcasted_iota(jnp.int32, (nout, be), 0)

    def body(k, acc):
        src = e_ref[0:1, pl.ds(k * be, be)]
        dst = e_ref[1:2, pl.ds(k * be, be)]
        ost = (iota_g == src).astype(jnp.bfloat16)
        odt = (iota_s == dst).astype(jnp.bfloat16)
        xg = lax.dot_general(ost, xb, _C00,
                             preferred_element_type=jnp.float32)
        return acc + jnp.dot(odt, xg.astype(jnp.bfloat16),
                             preferred_element_type=jnp.float32)

    o_ref[...] = lax.fori_loop(0, eh // be, body,
                               jnp.zeros((nout, d2), jnp.float32))


def _run_edge_agg(edges3, xb, nout):
    eh = edges3.shape[2]
    d2 = xb.shape[1]
    return pl.pallas_call(
        _edge_agg_kernel,
        out_shape=jax.ShapeDtypeStruct((2, nout, d2), jnp.float32),
        grid=(2,),
        in_specs=[
            pl.BlockSpec((None, 2, eh), lambda i: (i, 0, 0)),
            pl.BlockSpec((xb.shape[0], d2), lambda i: (0, 0)),
        ],
        out_specs=pl.BlockSpec((None, nout, d2), lambda i: (i, 0, 0)),
        compiler_params=pltpu.CompilerParams(
            dimension_semantics=("parallel",)),
    )(edges3, xb)


def _cgin_cls_kernel(ap_ref, xr_ref, w_ref, v_ref, cw_ref, cb_ref, o_ref):
    ap = ap_ref[0] + ap_ref[1]
    agg = ap[:, :_D] + ap[:, _D:]
    h = xr_ref[...] + agg
    w = w_ref[...]
    v = v_ref[...]
    h1 = jnp.maximum(
        jnp.dot(h, w[0], preferred_element_type=jnp.float32) + v[0, 0:1], 0.0)
    h2 = jnp.dot(h1, w[1], preferred_element_type=jnp.float32) + v[0, 1:2]
    xo = jnp.maximum(h2 * v[0, 2:3] + v[0, 3:4], 0.0)
    o_ref[...] = (jnp.dot(xo, cw_ref[...], preferred_element_type=jnp.float32)
                  + cb_ref[...])


def _run_branches(idx_s, ntab_s, etab_s, w_s, v_s):
    n = idx_s.shape[2]
    return pl.pallas_call(
        _branch_kernel,
        out_shape=jax.ShapeDtypeStruct((3, _G, _D), jnp.float32),
        grid=(3,),
        in_specs=[
            pl.BlockSpec((None, 16, n), lambda b: (b, 0, 0)),
            pl.BlockSpec((None, _NVOC, 2 * _D), lambda b: (b, 0, 0)),
            pl.BlockSpec((None, _EVOC, 2 * _D), lambda b: (b, 0, 0)),
            pl.BlockSpec((None, 4, _D, _D), lambda b: (b, 0, 0, 0)),
            pl.BlockSpec((None, 2, 4, _D), lambda b: (b, 0, 0, 0)),
        ],
        out_specs=pl.BlockSpec((None, _G, _D), lambda b: (b, 0, 0)),
        compiler_params=pltpu.CompilerParams(
            dimension_semantics=("parallel",)),
    )(idx_s, ntab_s, etab_s, w_s, v_s)


def _run_cgin_classifier(x, agg_part, h_w, h_v, cw, cb):
    nout = agg_part.shape[1]
    d2 = agg_part.shape[2]
    blk = nout // 2
    c = cw.shape[1]
    return pl.pallas_call(
        _cgin_cls_kernel,
        out_shape=jax.ShapeDtypeStruct((nout, c), jnp.float32),
        grid=(2,),
        in_specs=[
            pl.BlockSpec((2, blk, d2), lambda i: (0, i, 0)),
            pl.BlockSpec((blk, _D), lambda i: (i + 1, 0)),
            pl.BlockSpec((2, _D, _D), lambda i: (0, 0, 0)),
            pl.BlockSpec((1, 4, _D), lambda i: (0, 0, 0)),
            pl.BlockSpec((_D, c), lambda i: (0, 0)),
            pl.BlockSpec((1, c), lambda i: (0, 0)),
        ],
        out_specs=pl.BlockSpec((blk, c), lambda i: (i, 0)),
        compiler_params=pltpu.CompilerParams(
            dimension_semantics=("parallel",)),
    )(agg_part, x, h_w, h_v, cw, cb)


def _branch_idx_rows(node_idx_rows, edge_idx_rows, src, dst, batch, n):
    pad = jnp.zeros((1, n), jnp.int32)
    return jnp.concatenate(
        [node_idx_rows, edge_idx_rows,
         src[None, :].astype(jnp.int32), dst[None, :].astype(jnp.int32),
         batch[None, :].astype(jnp.int32), pad], axis=0)


def kernel(m_node0, m_edge0, m_W, m_V, r1_node0, r1_node1, r1_node2, r1_node3,
           r1_node4, r1_node5, r1_node6, r1_node7, r1_node8, r1_edge0,
           r1_edge1, r1_edge2, r1_W, r1_V, r2_node0, r2_node1, r2_node2,
           r2_node3, r2_node4, r2_node5, r2_node6, r2_node7, r2_node8,
           r2_edge0, r2_edge1, r2_edge2, r2_W, r2_V, h_W, h_V, cw, cb,
           motif_x, motif_edge_index, motif_edge_attr, motif_batch,
           raw_x_1, raw_edge_index_1, raw_edge_attr_1, raw_batch_1,
           raw_x_2, raw_edge_index_2, raw_edge_attr_2, raw_batch_2,
           heter_edge_index):
    n = motif_x.shape[0]
    e = motif_edge_index.shape[1]

    def _pad_rows(t, h):
        padded = jnp.concatenate(
            [t, jnp.zeros((h - t.shape[0], _D), jnp.float32)], axis=0)
        th = padded.astype(jnp.bfloat16).astype(jnp.float32)
        tl = (padded - th).astype(jnp.bfloat16).astype(jnp.float32)
        return jnp.concatenate([th, tl], axis=1)

    ntab_m = _pad_rows(m_node0, _NVOC)
    ntab_r1 = _pad_rows(jnp.concatenate(
        [r1_node0, r1_node1, r1_node2, r1_node3, r1_node4, r1_node5,
         r1_node6, r1_node7, r1_node8], axis=0), _NVOC)
    ntab_r2 = _pad_rows(jnp.concatenate(
        [r2_node0, r2_node1, r2_node2, r2_node3, r2_node4, r2_node5,
         r2_node6, r2_node7, r2_node8], axis=0), _NVOC)
    ntab_s = jnp.stack([ntab_m, ntab_r1, ntab_r2])

    etab_m = _pad_rows(m_edge0, _EVOC)
    etab_r1 = _pad_rows(jnp.concatenate([r1_edge0, r1_edge1, r1_edge2],
                                        axis=0), _EVOC)
    etab_r2 = _pad_rows(jnp.concatenate([r2_edge0, r2_edge1, r2_edge2],
                                        axis=0), _EVOC)
    etab_s = jnp.stack([etab_m, etab_r1, etab_r2])

    node_off = jnp.array(_NODE_OFF, jnp.int32)[:, None]
    edge_off = jnp.array(_EDGE_OFF, jnp.int32)[:, None]
    pad_n = jnp.full((8, n), _NODE_PAD_IDX, jnp.int32)
    m_nrows = jnp.concatenate([motif_x[None, :].astype(jnp.int32), pad_n],
                              axis=0)
    r1_nrows = raw_x_1.T.astype(jnp.int32) + node_off
    r2_nrows = raw_x_2.T.astype(jnp.int32) + node_off
    pad_e = jnp.full((2, e), _EDGE_PAD_IDX, jnp.int32)
    m_erows = jnp.concatenate(
        [motif_edge_attr[None, :].astype(jnp.int32), pad_e], axis=0)
    r1_erows = raw_edge_attr_1.T.astype(jnp.int32) + edge_off
    r2_erows = raw_edge_attr_2.T.astype(jnp.int32) + edge_off

    idx_s = jnp.stack([
        _branch_idx_rows(m_nrows, m_erows, motif_edge_index[0],
                         motif_edge_index[1], motif_batch, n),
        _branch_idx_rows(r1_nrows, r1_erows, raw_edge_index_1[0],
                         raw_edge_index_1[1], raw_batch_1, n),
        _branch_idx_rows(r2_nrows, r2_erows, raw_edge_index_2[0],
                         raw_edge_index_2[1], raw_batch_2, n),
    ])

    w_s = jnp.stack([m_W, r1_W, r2_W])
    v_s = jnp.stack([m_V, r1_V, r2_V])

    branch_out = _run_branches(idx_s, ntab_s, etab_s, w_s, v_s)
    node_feature = branch_out.reshape(3 * _G, _D)

    nout = 2 * _G
    het_e = heter_edge_index.shape[1]
    src = heter_edge_index[0].astype(jnp.int32)
    dstp = heter_edge_index[1].astype(jnp.int32) - _G
    dstp = jnp.where(dstp >= 0, dstp, nout + 7)
    edges3 = jnp.stack([src.reshape(2, het_e // 2),
                        dstp.reshape(2, het_e // 2)], axis=1)
    xh = node_feature.astype(jnp.bfloat16)
    xl = (node_feature - xh.astype(jnp.float32)).astype(jnp.bfloat16)
    xcat = jnp.concatenate([xh, xl], axis=1)
    agg_part = _run_edge_agg(edges3, xcat, nout)

    logits = _run_cgin_classifier(node_feature, agg_part, h_W, h_V, cw, cb)

    pred1 = logits[0:_G, 0:1]
    pred2 = logits[_G:2 * _G, 1:3]
    return pred1, pred2


# speedup vs baseline: 1.5021x; 1.5021x over previous
import numpy as np
import jax
import jax.numpy as jnp
from jax import lax
from jax.experimental import pallas as pl
from jax.experimental.pallas import tpu as pltpu

_D = 32
_G = 512
_NVOC = 184
_EVOC = 24
_NODE_OFF = (0, 120, 125, 137, 149, 159, 165, 172, 174)
_EDGE_OFF = (0, 15, 21)
_NODE_PAD_IDX = 176
_EDGE_PAD_IDX = 23

_C00 = (((0,), (0,)), ((), ()))


def _branch_kernel(idx_ref, ntab_ref, etab_ref, w_ref, v_ref, o_ref):
    n = idx_ref.shape[1]
    e = n
    g = o_ref.shape[0]
    idx = idx_ref[...]
    w = w_ref[...]
    v = v_ref[...]
    n_layers = v.shape[0]

    iota_nv = lax.broadcasted_iota(jnp.int32, (_NVOC, n), 0)
    nht = (iota_nv == idx[0:1, :]).astype(jnp.float32)
    for i in range(1, 9):
        nht = nht + (iota_nv == idx[i:i + 1, :]).astype(jnp.float32)
    x2 = lax.dot_general(nht, ntab_ref[...], _C00,
                         preferred_element_type=jnp.float32)
    x = x2[:, :_D] + x2[:, _D:]

    iota_ev = lax.broadcasted_iota(jnp.int32, (_EVOC, e), 0)
    eht = (iota_ev == idx[9:10, :]).astype(jnp.float32)
    for i in range(10, 12):
        eht = eht + (iota_ev == idx[i:i + 1, :]).astype(jnp.float32)
    ea2 = lax.dot_general(eht, etab_ref[...], _C00,
                          preferred_element_type=jnp.float32)
    ea = ea2[:, :_D] + ea2[:, _D:]

    iota_ne = lax.broadcasted_iota(jnp.int32, (n, e), 0)
    ost = (iota_ne == idx[12:13, :]).astype(jnp.float32)
    odt = (iota_ne == idx[13:14, :]).astype(jnp.float32)
    iota_gn = lax.broadcasted_iota(jnp.int32, (g, n), 0)
    pm = (iota_gn == idx[14:15, :]).astype(jnp.float32)
    pm = pm / jnp.maximum(jnp.sum(pm, axis=1, keepdims=True), 1.0)

    for l in range(n_layers):
        xg = lax.dot_general(ost, x, _C00,
                             preferred_element_type=jnp.float32)
        msg = jnp.maximum(xg + ea, 0.0)
        agg = jnp.dot(odt, msg, preferred_element_type=jnp.float32)
        h = x + agg
        h1 = jnp.maximum(
            jnp.dot(h, w[2 * l], preferred_element_type=jnp.float32)
            + v[l, 0:1], 0.0)
        h2 = jnp.dot(h1, w[2 * l + 1],
                     preferred_element_type=jnp.float32) + v[l, 1:2]
        x = jnp.maximum(h2 * v[l, 2:3] + v[l, 3:4], 0.0)

    o_ref[...] = jnp.dot(pm, x, preferred_element_type=jnp.float32)


def _edge_agg_kernel(e_ref, xb_ref, o_ref):
    eh = e_ref.shape[1]
    nfull = xb_ref.shape[0]
    nout = o_ref.shape[0]
    d2 = xb_ref.shape[1]
    be = 1024 if eh % 1024 == 0 else eh
    xb = xb_ref[...]
    iota_g = lax.broadcasted_iota(jnp.int32, (nfull, be), 0)
    iota_s = lax.broadcasted_iota(jnp.int32, (nout, be), 0)

    def body(k, acc):
        src = e_ref[0:1, pl.ds(k * be, be)]
        dst = e_ref[1:2, pl.ds(k * be, be)]
        ost = (iota_g == src).astype(jnp.bfloat16)
        odt = (iota_s == dst).astype(jnp.bfloat16)
        xg = lax.dot_general(ost, xb, _C00,
                             preferred_element_type=jnp.float32)
        return acc + jnp.dot(odt, xg.astype(jnp.bfloat16),
                             preferred_element_type=jnp.float32)

    o_ref[...] = lax.fori_loop(0, eh // be, body,
                               jnp.zeros((nout, d2), jnp.float32))


def _run_edge_agg(edges3, xb, nout):
    eh = edges3.shape[2]
    d2 = xb.shape[1]
    return pl.pallas_call(
        _edge_agg_kernel,
        out_shape=jax.ShapeDtypeStruct((2, nout, d2), jnp.float32),
        grid=(2,),
        in_specs=[
            pl.BlockSpec((None, 2, eh), lambda i: (i, 0, 0)),
            pl.BlockSpec((xb.shape[0], d2), lambda i: (0, 0)),
        ],
        out_specs=pl.BlockSpec((None, nout, d2), lambda i: (i, 0, 0)),
        compiler_params=pltpu.CompilerParams(
            dimension_semantics=("parallel",)),
    )(edges3, xb)


def _cgin_cls_kernel(a_ref, xf_ref, ap_ref, xr_ref, w_ref, v_ref, cw_ref,
                     cb_ref, o_ref):
    ap = ap_ref[0] + ap_ref[1]
    agg = (jnp.dot(a_ref[...], xf_ref[...],
                   preferred_element_type=jnp.float32)
           + ap[:, :_D] + ap[:, _D:])
    h = xr_ref[...] + agg
    w = w_ref[...]
    v = v_ref[...]
    h1 = jnp.maximum(
        jnp.dot(h, w[0], preferred_element_type=jnp.float32) + v[0, 0:1], 0.0)
    h2 = jnp.dot(h1, w[1], preferred_element_type=jnp.float32) + v[0, 1:2]
    xo = jnp.maximum(h2 * v[0, 2:3] + v[0, 3:4], 0.0)
    o_ref[...] = (jnp.dot(xo, cw_ref[...], preferred_element_type=jnp.float32)
                  + cb_ref[...])


def _run_branches(idx_s, ntab_s, etab_s, w_s, v_s):
    n = idx_s.shape[2]
    return pl.pallas_call(
        _branch_kernel,
        out_shape=jax.ShapeDtypeStruct((3, _G, _D), jnp.float32),
        grid=(3,),
        in_specs=[
            pl.BlockSpec((None, 16, n), lambda b: (b, 0, 0)),
            pl.BlockSpec((None, _NVOC, 2 * _D), lambda b: (b, 0, 0)),
            pl.BlockSpec((None, _EVOC, 2 * _D), lambda b: (b, 0, 0)),
            pl.BlockSpec((None, 4, _D, _D), lambda b: (b, 0, 0, 0)),
            pl.BlockSpec((None, 2, 4, _D), lambda b: (b, 0, 0, 0)),
        ],
        out_specs=pl.BlockSpec((None, _G, _D), lambda b: (b, 0, 0)),
        compiler_params=pltpu.CompilerParams(
            dimension_semantics=("parallel",)),
    )(idx_s, ntab_s, etab_s, w_s, v_s)


def _run_cgin_classifier(x, adj, agg_part, h_w, h_v, cw, cb):
    nh = x.shape[0]
    nout = agg_part.shape[1]
    d2 = agg_part.shape[2]
    blk = nout // 2
    c = cw.shape[1]
    return pl.pallas_call(
        _cgin_cls_kernel,
        out_shape=jax.ShapeDtypeStruct((nout, c), jnp.float32),
        grid=(2,),
        in_specs=[
            pl.BlockSpec((blk, nh), lambda i: (i, 0)),
            pl.BlockSpec((nh, _D), lambda i: (0, 0)),
            pl.BlockSpec((2, blk, d2), lambda i: (0, i, 0)),
            pl.BlockSpec((blk, _D), lambda i: (i + 1, 0)),
            pl.BlockSpec((2, _D, _D), lambda i: (0, 0, 0)),
            pl.BlockSpec((1, 4, _D), lambda i: (0, 0, 0)),
            pl.BlockSpec((_D, c), lambda i: (0, 0)),
            pl.BlockSpec((1, c), lambda i: (0, 0)),
        ],
        out_specs=pl.BlockSpec((blk, c), lambda i: (i, 0)),
        compiler_params=pltpu.CompilerParams(
            dimension_semantics=("parallel",)),
    )(adj, x, agg_part, x, h_w, h_v, cw, cb)


def _branch_idx_rows(node_idx_rows, edge_idx_rows, src, dst, batch, n):
    pad = jnp.zeros((1, n), jnp.int32)
    return jnp.concatenate(
        [node_idx_rows, edge_idx_rows,
         src[None, :].astype(jnp.int32), dst[None, :].astype(jnp.int32),
         batch[None, :].astype(jnp.int32), pad], axis=0)


def kernel(m_node0, m_edge0, m_W, m_V, r1_node0, r1_node1, r1_node2, r1_node3,
           r1_node4, r1_node5, r1_node6, r1_node7, r1_node8, r1_edge0,
           r1_edge1, r1_edge2, r1_W, r1_V, r2_node0, r2_node1, r2_node2,
           r2_node3, r2_node4, r2_node5, r2_node6, r2_node7, r2_node8,
           r2_edge0, r2_edge1, r2_edge2, r2_W, r2_V, h_W, h_V, cw, cb,
           motif_x, motif_edge_index, motif_edge_attr, motif_batch,
           raw_x_1, raw_edge_index_1, raw_edge_attr_1, raw_batch_1,
           raw_x_2, raw_edge_index_2, raw_edge_attr_2, raw_batch_2,
           heter_edge_index):
    n = motif_x.shape[0]
    e = motif_edge_index.shape[1]

    def _pad_rows(t, h):
        padded = jnp.concatenate(
            [t, jnp.zeros((h - t.shape[0], _D), jnp.float32)], axis=0)
        th = padded.astype(jnp.bfloat16).astype(jnp.float32)
        tl = (padded - th).astype(jnp.bfloat16).astype(jnp.float32)
        return jnp.concatenate([th, tl], axis=1)

    ntab_m = _pad_rows(m_node0, _NVOC)
    ntab_r1 = _pad_rows(jnp.concatenate(
        [r1_node0, r1_node1, r1_node2, r1_node3, r1_node4, r1_node5,
         r1_node6, r1_node7, r1_node8], axis=0), _NVOC)
    ntab_r2 = _pad_rows(jnp.concatenate(
        [r2_node0, r2_node1, r2_node2, r2_node3, r2_node4, r2_node5,
         r2_node6, r2_node7, r2_node8], axis=0), _NVOC)
    ntab_s = jnp.stack([ntab_m, ntab_r1, ntab_r2])

    etab_m = _pad_rows(m_edge0, _EVOC)
    etab_r1 = _pad_rows(jnp.concatenate([r1_edge0, r1_edge1, r1_edge2],
                                        axis=0), _EVOC)
    etab_r2 = _pad_rows(jnp.concatenate([r2_edge0, r2_edge1, r2_edge2],
                                        axis=0), _EVOC)
    etab_s = jnp.stack([etab_m, etab_r1, etab_r2])

    node_off = jnp.array(_NODE_OFF, jnp.int32)[:, None]
    edge_off = jnp.array(_EDGE_OFF, jnp.int32)[:, None]
    pad_n = jnp.full((8, n), _NODE_PAD_IDX, jnp.int32)
    m_nrows = jnp.concatenate([motif_x[None, :].astype(jnp.int32), pad_n],
                              axis=0)
    r1_nrows = raw_x_1.T.astype(jnp.int32) + node_off
    r2_nrows = raw_x_2.T.astype(jnp.int32) + node_off
    pad_e = jnp.full((2, e), _EDGE_PAD_IDX, jnp.int32)
    m_erows = jnp.concatenate(
        [motif_edge_attr[None, :].astype(jnp.int32), pad_e], axis=0)
    r1_erows = raw_edge_attr_1.T.astype(jnp.int32) + edge_off
    r2_erows = raw_edge_attr_2.T.astype(jnp.int32) + edge_off

    idx_s = jnp.stack([
        _branch_idx_rows(m_nrows, m_erows, motif_edge_index[0],
                         motif_edge_index[1], motif_batch, n),
        _branch_idx_rows(r1_nrows, r1_erows, raw_edge_index_1[0],
                         raw_edge_index_1[1], raw_batch_1, n),
        _branch_idx_rows(r2_nrows, r2_erows, raw_edge_index_2[0],
                         raw_edge_index_2[1], raw_batch_2, n),
    ])

    w_s = jnp.stack([m_W, r1_W, r2_W])
    v_s = jnp.stack([m_V, r1_V, r2_V])

    branch_out = _run_branches(idx_s, ntab_s, etab_s, w_s, v_s)
    node_feature = branch_out.reshape(3 * _G, _D)

    nout = 2 * _G
    nh = 3 * _G
    het_e = heter_edge_index.shape[1]
    src = heter_edge_index[0].astype(jnp.int32)
    dstp = heter_edge_index[1].astype(jnp.int32) - _G
    dstp = jnp.where(dstp >= 0, dstp, nout + 7)

    esc = het_e // 2
    adj = jnp.zeros((nout, nh), jnp.float32).at[dstp[:esc], src[:esc]].add(
        1.0, mode="drop")
    etc = het_e - esc
    edges3 = jnp.stack([src[esc:].reshape(2, etc // 2),
                        dstp[esc:].reshape(2, etc // 2)], axis=1)
    xh = node_feature.astype(jnp.bfloat16)
    xl = (node_feature - xh.astype(jnp.float32)).astype(jnp.bfloat16)
    xcat = jnp.concatenate([xh, xl], axis=1)
    agg_part = _run_edge_agg(edges3, xcat, nout)

    logits = _run_cgin_classifier(node_feature, adj, agg_part, h_W, h_V,
                                  cw, cb)

    pred1 = logits[0:_G, 0:1]
    pred2 = logits[_G:2 * _G, 1:3]
    return pred1, pred2


# speedup vs baseline: 1.5028x; 1.0005x over previous
import numpy as np
import jax
import jax.numpy as jnp
from jax import lax
from jax.experimental import pallas as pl
from jax.experimental.pallas import tpu as pltpu

_D = 32
_G = 512
_NVOC = 184
_EVOC = 24
_NODE_OFF = (0, 120, 125, 137, 149, 159, 165, 172, 174)
_EDGE_OFF = (0, 15, 21)
_NODE_PAD_IDX = 176
_EDGE_PAD_IDX = 23

_C00 = (((0,), (0,)), ((), ()))


def _branch_kernel(idx_ref, ntab_ref, etab_ref, w_ref, v_ref, o_ref):
    n = idx_ref.shape[1]
    e = n
    g = o_ref.shape[0]
    idx = idx_ref[...]
    w = w_ref[...]
    v = v_ref[...]
    n_layers = v.shape[0]

    iota_nv = lax.broadcasted_iota(jnp.int32, (_NVOC, n), 0)
    nht = (iota_nv == idx[0:1, :]).astype(jnp.float32)
    for i in range(1, 9):
        nht = nht + (iota_nv == idx[i:i + 1, :]).astype(jnp.float32)
    x2 = lax.dot_general(nht, ntab_ref[...], _C00,
                         preferred_element_type=jnp.float32)
    x = x2[:, :_D] + x2[:, _D:2 * _D] + x2[:, 2 * _D:]

    iota_ev = lax.broadcasted_iota(jnp.int32, (_EVOC, e), 0)
    eht = (iota_ev == idx[9:10, :]).astype(jnp.float32)
    for i in range(10, 12):
        eht = eht + (iota_ev == idx[i:i + 1, :]).astype(jnp.float32)
    ea2 = lax.dot_general(eht, etab_ref[...], _C00,
                          preferred_element_type=jnp.float32)
    ea = ea2[:, :_D] + ea2[:, _D:2 * _D] + ea2[:, 2 * _D:]

    iota_ne = lax.broadcasted_iota(jnp.int32, (n, e), 0)
    ost = (iota_ne == idx[12:13, :]).astype(jnp.float32)
    odt = (iota_ne == idx[13:14, :]).astype(jnp.float32)
    iota_gn = lax.broadcasted_iota(jnp.int32, (g, n), 0)
    pm = (iota_gn == idx[14:15, :]).astype(jnp.float32)
    pm = pm / jnp.maximum(jnp.sum(pm, axis=1, keepdims=True), 1.0)

    for l in range(n_layers):
        xg = lax.dot_general(ost, x, _C00,
                             preferred_element_type=jnp.float32)
        msg = jnp.maximum(xg + ea, 0.0)
        agg = jnp.dot(odt, msg, preferred_element_type=jnp.float32)
        h = x + agg
        h1 = jnp.maximum(
            jnp.dot(h, w[2 * l], preferred_element_type=jnp.float32)
            + v[l, 0:1], 0.0)
        h2 = jnp.dot(h1, w[2 * l + 1],
                     preferred_element_type=jnp.float32) + v[l, 1:2]
        x = jnp.maximum(h2 * v[l, 2:3] + v[l, 3:4], 0.0)

    o_ref[...] = jnp.dot(pm, x, preferred_element_type=jnp.float32)


def _edge_agg_kernel(e_ref, xb_ref, o_ref):
    eh = e_ref.shape[1]
    nfull = xb_ref.shape[0]
    nout = o_ref.shape[0]
    d2 = xb_ref.shape[1]
    be = 1024 if eh % 1024 == 0 else eh
    xb = xb_ref[...]
    iota_g = lax.broadcasted_iota(jnp.int32, (nfull, be), 0)
    iota_s = lax.broadcasted_iota(jnp.int32, (nout, be), 0)

    def body(k, acc):
        src = e_ref[0:1, pl.ds(k * be, be)]
        dst = e_ref[1:2, pl.ds(k * be, be)]
        ost = (iota_g == src).astype(jnp.bfloat16)
        odt = (iota_s == dst).astype(jnp.bfloat16)
        xg = lax.dot_general(ost, xb, _C00,
                             preferred_element_type=jnp.float32)
        return acc + jnp.dot(odt, xg.astype(jnp.bfloat16),
                             preferred_element_type=jnp.float32)

    o_ref[...] = lax.fori_loop(0, eh // be, body,
                               jnp.zeros((nout, d2), jnp.float32))


def _run_edge_agg(edges3, xb, nout):
    eh = edges3.shape[2]
    d2 = xb.shape[1]
    return pl.pallas_call(
        _edge_agg_kernel,
        out_shape=jax.ShapeDtypeStruct((2, nout, d2), jnp.float32),
        grid=(2,),
        in_specs=[
            pl.BlockSpec((None, 2, eh), lambda i: (i, 0, 0)),
            pl.BlockSpec((xb.shape[0], d2), lambda i: (0, 0)),
        ],
        out_specs=pl.BlockSpec((None, nout, d2), lambda i: (i, 0, 0)),
        compiler_params=pltpu.CompilerParams(
            dimension_semantics=("parallel",)),
    )(edges3, xb)


def _cgin_cls_kernel(a_ref, xf_ref, ap_ref, xr_ref, w_ref, v_ref, cw_ref,
                     cb_ref, o_ref):
    agg = (ap_ref[0] + ap_ref[1]
           + jnp.dot(a_ref[...], xf_ref[...],
                     preferred_element_type=jnp.float32))
    h = xr_ref[...] + agg
    w = w_ref[...]
    v = v_ref[...]
    h1 = jnp.maximum(
        jnp.dot(h, w[0], preferred_element_type=jnp.float32) + v[0, 0:1], 0.0)
    h2 = jnp.dot(h1, w[1], preferred_element_type=jnp.float32) + v[0, 1:2]
    xo = jnp.maximum(h2 * v[0, 2:3] + v[0, 3:4], 0.0)
    o_ref[...] = (jnp.dot(xo, cw_ref[...], preferred_element_type=jnp.float32)
                  + cb_ref[...])


def _run_branches(idx_s, ntab_s, etab_s, w_s, v_s):
    n = idx_s.shape[2]
    return pl.pallas_call(
        _branch_kernel,
        out_shape=jax.ShapeDtypeStruct((3, _G, _D), jnp.float32),
        grid=(3,),
        in_specs=[
            pl.BlockSpec((None, 16, n), lambda b: (b, 0, 0)),
            pl.BlockSpec((None, _NVOC, 3 * _D), lambda b: (b, 0, 0)),
            pl.BlockSpec((None, _EVOC, 3 * _D), lambda b: (b, 0, 0)),
            pl.BlockSpec((None, 4, _D, _D), lambda b: (b, 0, 0, 0)),
            pl.BlockSpec((None, 2, 4, _D), lambda b: (b, 0, 0, 0)),
        ],
        out_specs=pl.BlockSpec((None, _G, _D), lambda b: (b, 0, 0)),
        compiler_params=pltpu.CompilerParams(
            dimension_semantics=("parallel",)),
    )(idx_s, ntab_s, etab_s, w_s, v_s)


def _run_cgin_classifier(x, xsplit, adj, agg_part, h_w, h_v, cw, cb):
    nh = x.shape[0]
    nout = agg_part.shape[1]
    d2 = agg_part.shape[2]
    blk = nout // 2
    c = cw.shape[1]
    return pl.pallas_call(
        _cgin_cls_kernel,
        out_shape=jax.ShapeDtypeStruct((nout, c), jnp.float32),
        grid=(2,),
        in_specs=[
            pl.BlockSpec((blk, nh), lambda i: (i, 0)),
            pl.BlockSpec((nh, _D), lambda i: (0, 0)),
            pl.BlockSpec((2, blk, d2), lambda i: (0, i, 0)),
            pl.BlockSpec((blk, _D), lambda i: (i + 1, 0)),
            pl.BlockSpec((2, _D, _D), lambda i: (0, 0, 0)),
            pl.BlockSpec((1, 4, _D), lambda i: (0, 0, 0)),
            pl.BlockSpec((_D, c), lambda i: (0, 0)),
            pl.BlockSpec((1, c), lambda i: (0, 0)),
        ],
        out_specs=pl.BlockSpec((blk, c), lambda i: (i, 0)),
        compiler_params=pltpu.CompilerParams(
            dimension_semantics=("parallel",)),
    )(adj, xsplit, agg_part, x, h_w, h_v, cw, cb)


def _branch_idx_rows(node_idx_rows, edge_idx_rows, src, dst, batch, n):
    pad = jnp.zeros((1, n), jnp.int32)
    return jnp.concatenate(
        [node_idx_rows, edge_idx_rows,
         src[None, :].astype(jnp.int32), dst[None, :].astype(jnp.int32),
         batch[None, :].astype(jnp.int32), pad], axis=0)


def kernel(m_node0, m_edge0, m_W, m_V, r1_node0, r1_node1, r1_node2, r1_node3,
           r1_node4, r1_node5, r1_node6, r1_node7, r1_node8, r1_edge0,
           r1_edge1, r1_edge2, r1_W, r1_V, r2_node0, r2_node1, r2_node2,
           r2_node3, r2_node4, r2_node5, r2_node6, r2_node7, r2_node8,
           r2_edge0, r2_edge1, r2_edge2, r2_W, r2_V, h_W, h_V, cw, cb,
           motif_x, motif_edge_index, motif_edge_attr, motif_batch,
           raw_x_1, raw_edge_index_1, raw_edge_attr_1, raw_batch_1,
           raw_x_2, raw_edge_index_2, raw_edge_attr_2, raw_batch_2,
           heter_edge_index):
    n = motif_x.shape[0]
    e = motif_edge_index.shape[1]

    def _pad_rows(t, h):
        padded = jnp.concatenate(
            [t, jnp.zeros((h - t.shape[0], _D), jnp.float32)], axis=0)
        th = padded.astype(jnp.bfloat16).astype(jnp.float32)
        r1 = padded - th
        tl = r1.astype(jnp.bfloat16).astype(jnp.float32)
        tl2 = (r1 - tl).astype(jnp.bfloat16).astype(jnp.float32)
        return jnp.concatenate([th, tl, tl2], axis=1)

    ntab_m = _pad_rows(m_node0, _NVOC)
    ntab_r1 = _pad_rows(jnp.concatenate(
        [r1_node0, r1_node1, r1_node2, r1_node3, r1_node4, r1_node5,
         r1_node6, r1_node7, r1_node8], axis=0), _NVOC)
    ntab_r2 = _pad_rows(jnp.concatenate(
        [r2_node0, r2_node1, r2_node2, r2_node3, r2_node4, r2_node5,
         r2_node6, r2_node7, r2_node8], axis=0), _NVOC)
    ntab_s = jnp.stack([ntab_m, ntab_r1, ntab_r2])

    etab_m = _pad_rows(m_edge0, _EVOC)
    etab_r1 = _pad_rows(jnp.concatenate([r1_edge0, r1_edge1, r1_edge2],
                                        axis=0), _EVOC)
    etab_r2 = _pad_rows(jnp.concatenate([r2_edge0, r2_edge1, r2_edge2],
                                        axis=0), _EVOC)
    etab_s = jnp.stack([etab_m, etab_r1, etab_r2])

    node_off = jnp.array(_NODE_OFF, jnp.int32)[:, None]
    edge_off = jnp.array(_EDGE_OFF, jnp.int32)[:, None]
    pad_n = jnp.full((8, n), _NODE_PAD_IDX, jnp.int32)
    m_nrows = jnp.concatenate([motif_x[None, :].astype(jnp.int32), pad_n],
                              axis=0)
    r1_nrows = raw_x_1.T.astype(jnp.int32) + node_off
    r2_nrows = raw_x_2.T.astype(jnp.int32) + node_off
    pad_e = jnp.full((2, e), _EDGE_PAD_IDX, jnp.int32)
    m_erows = jnp.concatenate(
        [motif_edge_attr[None, :].astype(jnp.int32), pad_e], axis=0)
    r1_erows = raw_edge_attr_1.T.astype(jnp.int32) + edge_off
    r2_erows = raw_edge_attr_2.T.astype(jnp.int32) + edge_off

    idx_s = jnp.stack([
        _branch_idx_rows(m_nrows, m_erows, motif_edge_index[0],
                         motif_edge_index[1], motif_batch, n),
        _branch_idx_rows(r1_nrows, r1_erows, raw_edge_index_1[0],
                         raw_edge_index_1[1], raw_batch_1, n),
        _branch_idx_rows(r2_nrows, r2_erows, raw_edge_index_2[0],
                         raw_edge_index_2[1], raw_batch_2, n),
    ])

    w_s = jnp.stack([m_W, r1_W, r2_W])
    v_s = jnp.stack([m_V, r1_V, r2_V])

    branch_out = _run_branches(idx_s, ntab_s, etab_s, w_s, v_s)
    node_feature = branch_out.reshape(3 * _G, _D)

    nout = 2 * _G
    nh = 3 * _G
    het_e = heter_edge_index.shape[1]
    src = heter_edge_index[0].astype(jnp.int32)
    dstp = heter_edge_index[1].astype(jnp.int32) - _G
    dstp = jnp.where(dstp >= 0, dstp, nout + 7)

    esc = het_e // 2
    adj = jnp.zeros((nout, nh), jnp.float32).at[dstp[:esc], src[:esc]].add(
        1.0, mode="drop")
    etc = het_e - esc
    edges3 = jnp.stack([src[esc:].reshape(2, etc // 2),
                        dstp[esc:].reshape(2, etc // 2)], axis=1)
    xb = node_feature.astype(jnp.bfloat16)
    agg_part = _run_edge_agg(edges3, xb, nout)

    logits = _run_cgin_classifier(node_feature, node_feature, adj, agg_part,
                                  h_W, h_V, cw, cb)

    pred1 = logits[0:_G, 0:1]
    pred2 = logits[_G:2 * _G, 1:3]
    return pred1, pred2
